# baseline (speedup 1.0000x reference)
import functools

import jax
import jax.numpy as jnp
from jax import lax
from jax.experimental import pallas as pl
from jax.experimental.pallas import tpu as pltpu
from jax.experimental.pallas import tpu_sc as plsc

SMOOTH = 8.0
VOCAB = 50000
Q = 512
N = 65536
D = 512

N_BLK = 2048
NW = 32
QPW = Q // NW
CH = 4096
L = 16
SM_BLK = 8


def _kern_body(h_ref, c_ref, o_ref):
    h = h_ref[...]
    c = c_ref[...]
    hf = h.astype(jnp.float32)
    cf = c.astype(jnp.float32)
    qsq = jnp.sum(hf * hf, axis=1, keepdims=True)
    ksq = jnp.sum(cf * cf, axis=1)[None, :]
    dots = lax.dot_general(
        h, c, dimension_numbers=(((1,), (1,)), ((), ())),
        preferred_element_type=jnp.float32)
    sq = jnp.maximum(qsq + ksq - 2.0 * dots, 0.0)
    o_ref[...] = jnp.exp(jnp.sqrt(sq) * (1.0 / SMOOTH))


def _kern_matrix(h_bf, c_bf):
    return pl.pallas_call(
        _kern_body,
        grid=(N // N_BLK,),
        in_specs=[
            pl.BlockSpec((Q, D), lambda i: (0, 0)),
            pl.BlockSpec((N_BLK, D), lambda i: (i, 0)),
        ],
        out_specs=pl.BlockSpec((Q, N_BLK), lambda i: (0, i)),
        out_shape=jax.ShapeDtypeStruct((Q, N), jnp.float32),
    )(h_bf, c_bf)


def _sc_scatter_body(kern_hbm, words_hbm, out_hbm, acc, wbuf, kbuf):
    wid = lax.axis_index("s") * 2 + lax.axis_index("c")
    pltpu.sync_copy(words_hbm, wbuf)

    def do_row(j):
        q = wid * QPW + j

        def zbody(i, _):
            acc[pl.ds(i * L, L)] = jnp.zeros((L,), jnp.float32)
            return 0
        lax.fori_loop(0, VOCAB // L, zbody, 0)

        for cb in range(N // CH):
            pltpu.sync_copy(kern_hbm.at[q, pl.ds(cb * CH, CH)], kbuf)

            def sbody(g, _):
                idx = wbuf[pl.ds(cb * CH + g * L, L)]
                val = kbuf[pl.ds(g * L, L)]
                plsc.addupdate_scatter(acc, [idx], val)
                return 0
            lax.fori_loop(0, CH // L, sbody, 0)

        pltpu.sync_copy(acc, out_hbm.at[q])

    for j in range(QPW):
        do_row(j)


def _sc_scatter(kern, words):
    mesh = plsc.VectorSubcoreMesh(core_axis_name="c", subcore_axis_name="s")
    f = pl.kernel(
        _sc_scatter_body,
        out_type=jax.ShapeDtypeStruct((Q, VOCAB), jnp.float32),
        mesh=mesh,
        scratch_types=[
            pltpu.VMEM((VOCAB,), jnp.float32),
            pltpu.VMEM((N,), jnp.int32),
            pltpu.VMEM((CH,), jnp.float32),
        ],
        compiler_params=pltpu.CompilerParams(needs_layout_passes=False),
    )
    return f(kern, words)


def _softmax_body(x_ref, o_ref):
    x = x_ref[...]
    m = jnp.max(x, axis=1, keepdims=True)
    s = jnp.sum(jnp.exp(x - m), axis=1, keepdims=True)
    o_ref[...] = x - (m + jnp.log(s))


def _log_softmax(x):
    return pl.pallas_call(
        _softmax_body,
        grid=(Q // SM_BLK,),
        in_specs=[pl.BlockSpec((SM_BLK, VOCAB), lambda i: (i, 0))],
        out_specs=pl.BlockSpec((SM_BLK, VOCAB), lambda i: (i, 0)),
        out_shape=jax.ShapeDtypeStruct((Q, VOCAB), jnp.float32),
    )(x)


def kernel(h_t, cache_hiddens, cache_words):
    h_bf = h_t.astype(jnp.bfloat16)
    c_bf = cache_hiddens.astype(jnp.bfloat16)
    kern = _kern_matrix(h_bf, c_bf)
    cache_p = _sc_scatter(kern, cache_words)
    return _log_softmax(cache_p)


# speedup vs baseline: 1.9307x; 1.9307x over previous
import functools

import jax
import jax.numpy as jnp
from jax import lax
from jax.experimental import pallas as pl
from jax.experimental.pallas import tpu as pltpu
from jax.experimental.pallas import tpu_sc as plsc

SMOOTH = 8.0
VOCAB = 50000
Q = 512
N = 65536
D = 512

N_BLK = 2048
NW = 32
QPW = Q // NW
CH = 4096
L = 16
SM_BLK = 8


def _kern_body(h_ref, c_ref, o_ref):
    h = h_ref[...]
    c = c_ref[...]
    hf = h.astype(jnp.float32)
    cf = c.astype(jnp.float32)
    qsq = jnp.sum(hf * hf, axis=1, keepdims=True)
    ksq = jnp.sum(cf * cf, axis=1)[None, :]
    dots = lax.dot_general(
        h, c, dimension_numbers=(((1,), (1,)), ((), ())),
        preferred_element_type=jnp.float32)
    sq = jnp.maximum(qsq + ksq - 2.0 * dots, 0.0)
    o_ref[...] = jnp.exp(jnp.sqrt(sq) * (1.0 / SMOOTH))


def _kern_matrix(h_bf, c_bf):
    return pl.pallas_call(
        _kern_body,
        grid=(N // N_BLK,),
        in_specs=[
            pl.BlockSpec((Q, D), lambda i: (0, 0)),
            pl.BlockSpec((N_BLK, D), lambda i: (i, 0)),
        ],
        out_specs=pl.BlockSpec((Q, N_BLK), lambda i: (0, i)),
        out_shape=jax.ShapeDtypeStruct((Q, N), jnp.float32),
    )(h_bf, c_bf)


def _sc_scatter_body(kern_hbm, words_hbm, out_hbm, acc, wbuf, kbuf,
                     semk0, semk1, semo):
    wid = lax.axis_index("s") * 2 + lax.axis_index("c")
    pltpu.sync_copy(words_hbm, wbuf)
    NCH = N // CH

    def scatter_chunk(base, buf_idx):
        @plsc.parallel_loop(0, CH // L, step=1, unroll=8)
        def s(g):
            idx = wbuf[pl.ds(base + g * L, L)]
            val = kbuf[buf_idx, pl.ds(g * L, L)]
            plsc.addupdate_scatter(acc, [idx], val)

    def row_body(j, _):
        q = wid * QPW + j
        h0 = pltpu.async_copy(
            kern_hbm.at[q, pl.ds(0, CH)], kbuf.at[0], semk0)

        @plsc.parallel_loop(0, VOCAB // L, step=1, unroll=8)
        def z(i):
            acc[pl.ds(i * L, L)] = jnp.zeros((L,), jnp.float32)

        def pair_body(p, _):
            c0 = 2 * p
            h1 = pltpu.async_copy(
                kern_hbm.at[q, pl.ds((c0 + 1) * CH, CH)], kbuf.at[1], semk1)
            pltpu.make_async_copy(
                kern_hbm.at[q, pl.ds(c0 * CH, CH)], kbuf.at[0], semk0).wait()
            scatter_chunk(c0 * CH, 0)

            @pl.when(c0 + 2 < NCH)
            def _():
                pltpu.async_copy(
                    kern_hbm.at[q, pl.ds((c0 + 2) * CH, CH)],
                    kbuf.at[0], semk0)
            h1.wait()
            scatter_chunk((c0 + 1) * CH, 1)
            return 0

        lax.fori_loop(0, NCH // 2, pair_body, 0)
        pltpu.sync_copy(acc, out_hbm.at[q])
        return 0

    lax.fori_loop(0, QPW, row_body, 0)


def _sc_scatter(kern, words):
    mesh = plsc.VectorSubcoreMesh(core_axis_name="c", subcore_axis_name="s")
    f = pl.kernel(
        _sc_scatter_body,
        out_type=jax.ShapeDtypeStruct((Q, VOCAB), jnp.float32),
        mesh=mesh,
        scratch_types=[
            pltpu.VMEM((VOCAB,), jnp.float32),
            pltpu.VMEM((N,), jnp.int32),
            pltpu.VMEM((2, CH), jnp.float32),
            pltpu.SemaphoreType.DMA,
            pltpu.SemaphoreType.DMA,
            pltpu.SemaphoreType.DMA,
        ],
        compiler_params=pltpu.CompilerParams(needs_layout_passes=False),
    )
    return f(kern, words)


def _softmax_body(x_ref, o_ref):
    x = x_ref[...]
    m = jnp.max(x, axis=1, keepdims=True)
    s = jnp.sum(jnp.exp(x - m), axis=1, keepdims=True)
    o_ref[...] = x - (m + jnp.log(s))


def _log_softmax(x):
    return pl.pallas_call(
        _softmax_body,
        grid=(Q // SM_BLK,),
        in_specs=[pl.BlockSpec((SM_BLK, VOCAB), lambda i: (i, 0))],
        out_specs=pl.BlockSpec((SM_BLK, VOCAB), lambda i: (i, 0)),
        out_shape=jax.ShapeDtypeStruct((Q, VOCAB), jnp.float32),
    )(x)


def kernel(h_t, cache_hiddens, cache_words):
    h_bf = h_t.astype(jnp.bfloat16)
    c_bf = cache_hiddens.astype(jnp.bfloat16)
    kern = _kern_matrix(h_bf, c_bf)
    cache_p = _sc_scatter(kern, cache_words)
    return _log_softmax(cache_p)


# speedup vs baseline: 2.3950x; 1.2405x over previous
import functools

import jax
import jax.numpy as jnp
from jax import lax
from jax.experimental import pallas as pl
from jax.experimental.pallas import tpu as pltpu
from jax.experimental.pallas import tpu_sc as plsc

SMOOTH = 8.0
VOCAB = 50000
Q = 512
N = 65536
D = 512

N_BLK = 2048
NW = 32
QH = Q // 2
QPW = QH // NW
CH = 4096
L = 16

LSE_BLK = 4096
LSE_STEPS = (VOCAB + LSE_BLK - 1) // LSE_BLK
TR_BLK = 1024
TR_STEPS = (VOCAB + TR_BLK - 1) // TR_BLK


def _kern_body(h_ref, c_ref, o_ref):
    h = h_ref[...]
    c = c_ref[...]
    qsq = jnp.sum(h * h, axis=1, keepdims=True)
    ksq = jnp.sum(c * c, axis=1)[None, :]
    dots = lax.dot_general(
        h.astype(jnp.bfloat16), c.astype(jnp.bfloat16),
        dimension_numbers=(((1,), (1,)), ((), ())),
        preferred_element_type=jnp.float32)
    sq = jnp.maximum(qsq + ksq - 2.0 * dots, 0.0)
    o_ref[...] = jnp.exp(jnp.sqrt(sq) * (1.0 / SMOOTH))


def _kern_matrix(h_half, cache_hiddens):
    return pl.pallas_call(
        _kern_body,
        grid=(N // N_BLK,),
        in_specs=[
            pl.BlockSpec((QH, D), lambda i: (0, 0)),
            pl.BlockSpec((N_BLK, D), lambda i: (i, 0)),
        ],
        out_specs=pl.BlockSpec((QH, N_BLK), lambda i: (0, i)),
        out_shape=jax.ShapeDtypeStruct((QH, N), jnp.float32),
    )(h_half, cache_hiddens)


def _sc_scatter_body(kern_hbm, words_hbm, out_hbm, acc, wbuf, kbuf,
                     semk0, semk1, semo):
    wid = lax.axis_index("s") * 2 + lax.axis_index("c")
    pltpu.sync_copy(words_hbm, wbuf)
    NCH = N // CH

    def scatter_chunk(base, buf_idx):
        @plsc.parallel_loop(0, CH // L, step=1, unroll=16)
        def s(g):
            idx = wbuf[pl.ds(base + g * L, L)]
            val = kbuf[buf_idx, pl.ds(g * L, L)]
            plsc.addupdate_scatter(acc, [idx], val)

    def row_body(j, _):
        q = wid * QPW + j
        pltpu.async_copy(kern_hbm.at[q, pl.ds(0, CH)], kbuf.at[0], semk0)

        @plsc.parallel_loop(0, VOCAB // L, step=1, unroll=16)
        def z(i):
            acc[pl.ds(i * L, L)] = jnp.zeros((L,), jnp.float32)

        def pair_body(p, _):
            c0 = 2 * p
            h1 = pltpu.async_copy(
                kern_hbm.at[q, pl.ds((c0 + 1) * CH, CH)], kbuf.at[1], semk1)
            pltpu.make_async_copy(
                kern_hbm.at[q, pl.ds(c0 * CH, CH)], kbuf.at[0], semk0).wait()
            scatter_chunk(c0 * CH, 0)

            @pl.when(c0 + 2 < NCH)
            def _():
                pltpu.async_copy(
                    kern_hbm.at[q, pl.ds((c0 + 2) * CH, CH)],
                    kbuf.at[0], semk0)
            h1.wait()
            scatter_chunk((c0 + 1) * CH, 1)
            return 0

        lax.fori_loop(0, NCH // 2, pair_body, 0)
        pltpu.sync_copy(acc, out_hbm.at[q])
        return 0

    lax.fori_loop(0, QPW, row_body, 0)


def _sc_scatter(kern, words):
    mesh = plsc.VectorSubcoreMesh(core_axis_name="c", subcore_axis_name="s")
    f = pl.kernel(
        _sc_scatter_body,
        out_type=jax.ShapeDtypeStruct((QH, VOCAB), jnp.float32),
        mesh=mesh,
        scratch_types=[
            pltpu.VMEM((VOCAB,), jnp.float32),
            pltpu.VMEM((N,), jnp.int32),
            pltpu.VMEM((2, CH), jnp.float32),
            pltpu.SemaphoreType.DMA,
            pltpu.SemaphoreType.DMA,
            pltpu.SemaphoreType.DMA,
        ],
        compiler_params=pltpu.CompilerParams(needs_layout_passes=False),
    )
    return f(kern, words)


def _lse_body(x_ref, m_ref, s_ref):
    i = pl.program_id(0)
    x = x_ref[...]
    col = i * LSE_BLK + lax.broadcasted_iota(jnp.int32, x.shape, 1)
    valid = col < VOCAB
    xm = jnp.where(valid, x, -jnp.inf)
    bm = jnp.max(xm, axis=1, keepdims=True)

    @pl.when(i == 0)
    def _():
        m_ref[...] = jnp.full_like(m_ref, -jnp.inf)
        s_ref[...] = jnp.zeros_like(s_ref)

    m_old = m_ref[...]
    m_new = jnp.maximum(m_old, bm)
    bs = jnp.sum(jnp.where(valid, jnp.exp(x - m_new), 0.0), axis=1,
                 keepdims=True)
    s_ref[...] = s_ref[...] * jnp.exp(m_old - m_new) + bs
    m_ref[...] = m_new


def _lse(x):
    return pl.pallas_call(
        _lse_body,
        grid=(LSE_STEPS,),
        in_specs=[pl.BlockSpec((QH, LSE_BLK), lambda i: (0, i))],
        out_specs=[
            pl.BlockSpec((QH, 1), lambda i: (0, 0)),
            pl.BlockSpec((QH, 1), lambda i: (0, 0)),
        ],
        out_shape=[
            jax.ShapeDtypeStruct((QH, 1), jnp.float32),
            jax.ShapeDtypeStruct((QH, 1), jnp.float32),
        ],
    )(x)


def _sub_transpose_body(xa_ref, xb_ref, ma_ref, sa_ref, mb_ref, sb_ref,
                        o_ref):
    xa = xa_ref[...]
    xb = xb_ref[...]
    offa = ma_ref[...] + jnp.log(sa_ref[...])
    offb = mb_ref[...] + jnp.log(sb_ref[...])
    o_ref[:, :QH] = (xa - offa).T
    o_ref[:, QH:] = (xb - offb).T


def _sub_transpose(xa, xb, ma, sa, mb, sb):
    vec = pl.BlockSpec((QH, 1), lambda i: (0, 0))
    half = pl.BlockSpec((QH, TR_BLK), lambda i: (0, i))
    return pl.pallas_call(
        _sub_transpose_body,
        grid=(TR_STEPS,),
        in_specs=[half, half, vec, vec, vec, vec],
        out_specs=pl.BlockSpec((TR_BLK, Q), lambda i: (i, 0)),
        out_shape=jax.ShapeDtypeStruct((VOCAB, Q), jnp.float32),
    )(xa, xb, ma, sa, mb, sb)


def kernel(h_t, cache_hiddens, cache_words):
    kern_a = _kern_matrix(h_t[:QH], cache_hiddens)
    kern_b = _kern_matrix(h_t[QH:], cache_hiddens)
    cp_a = _sc_scatter(kern_a, cache_words)
    cp_b = _sc_scatter(kern_b, cache_words)
    ma, sa = _lse(cp_a)
    mb, sb = _lse(cp_b)
    out_t = _sub_transpose(cp_a, cp_b, ma, sa, mb, sb)
    return out_t.T


# speedup vs baseline: 2.3973x; 1.0010x over previous
import functools

import jax
import jax.numpy as jnp
from jax import lax
from jax.experimental import pallas as pl
from jax.experimental.pallas import tpu as pltpu
from jax.experimental.pallas import tpu_sc as plsc

SMOOTH = 8.0
VOCAB = 50000
Q = 512
N = 65536
D = 512

N_BLK = 2048
NW = 32
QH = Q // 2
QPW = QH // NW
CH = 4096
L = 16

LSE_BLK = 4096
LSE_STEPS = (VOCAB + LSE_BLK - 1) // LSE_BLK
TR_BLK = 1024
TR_STEPS = (VOCAB + TR_BLK - 1) // TR_BLK


def _kern_body(h_ref, c_ref, o_ref):
    h = h_ref[...]
    c = c_ref[...]
    qsq = jnp.sum(h * h, axis=1, keepdims=True)
    ksq = jnp.sum(c * c, axis=1)[None, :]
    dots = lax.dot_general(
        h.astype(jnp.bfloat16), c.astype(jnp.bfloat16),
        dimension_numbers=(((1,), (1,)), ((), ())),
        preferred_element_type=jnp.float32)
    sq = jnp.maximum(qsq + ksq - 2.0 * dots, 0.0)
    o_ref[...] = jnp.exp(jnp.sqrt(sq) * (1.0 / SMOOTH))


def _kern_matrix(h_half, cache_hiddens):
    return pl.pallas_call(
        _kern_body,
        grid=(N // N_BLK,),
        in_specs=[
            pl.BlockSpec((QH, D), lambda i: (0, 0)),
            pl.BlockSpec((N_BLK, D), lambda i: (i, 0)),
        ],
        out_specs=pl.BlockSpec((QH, N_BLK), lambda i: (0, i)),
        out_shape=jax.ShapeDtypeStruct((QH, N), jnp.float32),
    )(h_half, cache_hiddens)


def _sc_scatter_body(kern_hbm, words_hbm, out_hbm, acc, wbuf, kbuf,
                     semk0, semk1, semo):
    wid = lax.axis_index("s") * 2 + lax.axis_index("c")
    pltpu.sync_copy(words_hbm, wbuf)
    NCH = N // CH

    def scatter_chunk(base, buf_idx):
        @plsc.parallel_loop(0, CH // L, step=1, unroll=16)
        def s(g):
            idx = wbuf[pl.ds(base + g * L, L)]
            val = kbuf[buf_idx, pl.ds(g * L, L)]
            plsc.addupdate_scatter(acc, [idx], val)

    def row_body(j, _):
        q = wid * QPW + j
        pltpu.async_copy(kern_hbm.at[q, pl.ds(0, CH)], kbuf.at[0], semk0)

        with jax.named_scope("zero"):
            @plsc.parallel_loop(0, VOCAB // L, step=1, unroll=16)
            def z(i):
                acc[pl.ds(i * L, L)] = jnp.zeros((L,), jnp.float32)

        def pair_body(p, _):
            c0 = 2 * p
            h1 = pltpu.async_copy(
                kern_hbm.at[q, pl.ds((c0 + 1) * CH, CH)], kbuf.at[1], semk1)
            pltpu.make_async_copy(
                kern_hbm.at[q, pl.ds(c0 * CH, CH)], kbuf.at[0], semk0).wait()
            scatter_chunk(c0 * CH, 0)

            @pl.when(c0 + 2 < NCH)
            def _():
                pltpu.async_copy(
                    kern_hbm.at[q, pl.ds((c0 + 2) * CH, CH)],
                    kbuf.at[0], semk0)
            h1.wait()
            scatter_chunk((c0 + 1) * CH, 1)
            return 0

        with jax.named_scope("chunks"):
            lax.fori_loop(0, NCH // 2, pair_body, 0)
        with jax.named_scope("wout"):
            pltpu.sync_copy(acc, out_hbm.at[q])
        return 0

    lax.fori_loop(0, QPW, row_body, 0)


def _sc_scatter(kern, words):
    mesh = plsc.VectorSubcoreMesh(core_axis_name="c", subcore_axis_name="s")
    f = pl.kernel(
        _sc_scatter_body,
        out_type=jax.ShapeDtypeStruct((QH, VOCAB), jnp.float32),
        mesh=mesh,
        scratch_types=[
            pltpu.VMEM((VOCAB,), jnp.float32),
            pltpu.VMEM((N,), jnp.int32),
            pltpu.VMEM((2, CH), jnp.float32),
            pltpu.SemaphoreType.DMA,
            pltpu.SemaphoreType.DMA,
            pltpu.SemaphoreType.DMA,
        ],
        compiler_params=pltpu.CompilerParams(needs_layout_passes=False),
    )
    return f(kern, words)


def _lse_body(x_ref, m_ref, s_ref):
    i = pl.program_id(0)
    x = x_ref[...]
    col = i * LSE_BLK + lax.broadcasted_iota(jnp.int32, x.shape, 1)
    valid = col < VOCAB
    xm = jnp.where(valid, x, -jnp.inf)
    bm = jnp.max(xm, axis=1, keepdims=True)

    @pl.when(i == 0)
    def _():
        m_ref[...] = jnp.full_like(m_ref, -jnp.inf)
        s_ref[...] = jnp.zeros_like(s_ref)

    m_old = m_ref[...]
    m_new = jnp.maximum(m_old, bm)
    bs = jnp.sum(jnp.where(valid, jnp.exp(x - m_new), 0.0), axis=1,
                 keepdims=True)
    s_ref[...] = s_ref[...] * jnp.exp(m_old - m_new) + bs
    m_ref[...] = m_new


def _lse(x):
    return pl.pallas_call(
        _lse_body,
        grid=(LSE_STEPS,),
        in_specs=[pl.BlockSpec((QH, LSE_BLK), lambda i: (0, i))],
        out_specs=[
            pl.BlockSpec((QH, 1), lambda i: (0, 0)),
            pl.BlockSpec((QH, 1), lambda i: (0, 0)),
        ],
        out_shape=[
            jax.ShapeDtypeStruct((QH, 1), jnp.float32),
            jax.ShapeDtypeStruct((QH, 1), jnp.float32),
        ],
    )(x)


def _sub_transpose_body(xa_ref, xb_ref, ma_ref, sa_ref, mb_ref, sb_ref,
                        o_ref):
    xa = xa_ref[...]
    xb = xb_ref[...]
    offa = ma_ref[...] + jnp.log(sa_ref[...])
    offb = mb_ref[...] + jnp.log(sb_ref[...])
    o_ref[:, :QH] = (xa - offa).T
    o_ref[:, QH:] = (xb - offb).T


def _sub_transpose(xa, xb, ma, sa, mb, sb):
    vec = pl.BlockSpec((QH, 1), lambda i: (0, 0))
    half = pl.BlockSpec((QH, TR_BLK), lambda i: (0, i))
    return pl.pallas_call(
        _sub_transpose_body,
        grid=(TR_STEPS,),
        in_specs=[half, half, vec, vec, vec, vec],
        out_specs=pl.BlockSpec((TR_BLK, Q), lambda i: (i, 0)),
        out_shape=jax.ShapeDtypeStruct((VOCAB, Q), jnp.float32),
    )(xa, xb, ma, sa, mb, sb)


def kernel(h_t, cache_hiddens, cache_words):
    kern_a = _kern_matrix(h_t[:QH], cache_hiddens)
    kern_b = _kern_matrix(h_t[QH:], cache_hiddens)
    cp_a = _sc_scatter(kern_a, cache_words)
    cp_b = _sc_scatter(kern_b, cache_words)
    ma, sa = _lse(cp_a)
    mb, sb = _lse(cp_b)
    out_t = _sub_transpose(cp_a, cp_b, ma, sa, mb, sb)
    return out_t.T


# speedup vs baseline: 2.5491x; 1.0633x over previous
import functools

import jax
import jax.numpy as jnp
from jax import lax
from jax.experimental import pallas as pl
from jax.experimental.pallas import tpu as pltpu
from jax.experimental.pallas import tpu_sc as plsc

SMOOTH = 8.0
VOCAB = 50000
Q = 512
N = 65536
D = 512

N_BLK = 2048
NW = 32
QH = Q // 2
QPW = QH // NW
CH = 4096
L = 16

LSE_BLK = 4096
LSE_STEPS = (VOCAB + LSE_BLK - 1) // LSE_BLK
TR_BLK = 2048
TR_STEPS = (VOCAB + TR_BLK - 1) // TR_BLK


def _kern_body(h_ref, c_ref, o_ref):
    h = h_ref[...]
    c = c_ref[...]
    qsq = jnp.sum(h * h, axis=1, keepdims=True)
    ksq = jnp.sum(c * c, axis=1)[None, :]
    dots = lax.dot_general(
        h.astype(jnp.bfloat16), c.astype(jnp.bfloat16),
        dimension_numbers=(((1,), (1,)), ((), ())),
        preferred_element_type=jnp.float32)
    sq = jnp.maximum(qsq + ksq - 2.0 * dots, 0.0)
    o_ref[...] = jnp.exp(jnp.sqrt(sq) * (1.0 / SMOOTH))


def _kern_matrix(h_half, cache_hiddens):
    return pl.pallas_call(
        _kern_body,
        grid=(N // N_BLK,),
        in_specs=[
            pl.BlockSpec((QH, D), lambda i: (0, 0)),
            pl.BlockSpec((N_BLK, D), lambda i: (i, 0)),
        ],
        out_specs=pl.BlockSpec((QH, N_BLK), lambda i: (0, i)),
        out_shape=jax.ShapeDtypeStruct((QH, N), jnp.float32),
    )(h_half, cache_hiddens)


def _sc_scatter_body(kern_hbm, words_hbm, out_hbm, acc, wbuf, kbuf,
                     semk0, semk1, semo):
    wid = lax.axis_index("s") * 2 + lax.axis_index("c")
    pltpu.sync_copy(words_hbm, wbuf)
    NCH = N // CH

    def scatter_chunk(base, buf_idx):
        @plsc.parallel_loop(0, CH // L, step=1, unroll=16)
        def s(g):
            idx = wbuf[pl.ds(base + g * L, L)]
            val = kbuf[buf_idx, pl.ds(g * L, L)]
            plsc.addupdate_scatter(acc, [idx], val)

    def row_body(j, _):
        q = wid * QPW + j
        pltpu.async_copy(kern_hbm.at[q, pl.ds(0, CH)], kbuf.at[0], semk0)

        @plsc.parallel_loop(0, VOCAB // L, step=1, unroll=16)
        def z(i):
            acc[pl.ds(i * L, L)] = jnp.zeros((L,), jnp.float32)

        def pair_body(p, _):
            c0 = 2 * p
            h1 = pltpu.async_copy(
                kern_hbm.at[q, pl.ds((c0 + 1) * CH, CH)], kbuf.at[1], semk1)
            pltpu.make_async_copy(
                kern_hbm.at[q, pl.ds(c0 * CH, CH)], kbuf.at[0], semk0).wait()
            scatter_chunk(c0 * CH, 0)

            @pl.when(c0 + 2 < NCH)
            def _():
                pltpu.async_copy(
                    kern_hbm.at[q, pl.ds((c0 + 2) * CH, CH)],
                    kbuf.at[0], semk0)
            h1.wait()
            scatter_chunk((c0 + 1) * CH, 1)
            return 0

        lax.fori_loop(0, NCH // 2, pair_body, 0)
        pltpu.sync_copy(acc, out_hbm.at[q])
        return 0

    lax.fori_loop(0, QPW, row_body, 0)


def _sc_scatter(kern, words):
    mesh = plsc.VectorSubcoreMesh(core_axis_name="c", subcore_axis_name="s")
    f = pl.kernel(
        _sc_scatter_body,
        out_type=jax.ShapeDtypeStruct((QH, VOCAB), jnp.float32),
        mesh=mesh,
        scratch_types=[
            pltpu.VMEM((VOCAB,), jnp.float32),
            pltpu.VMEM((N,), jnp.int32),
            pltpu.VMEM((2, CH), jnp.float32),
            pltpu.SemaphoreType.DMA,
            pltpu.SemaphoreType.DMA,
            pltpu.SemaphoreType.DMA,
        ],
        compiler_params=pltpu.CompilerParams(needs_layout_passes=False),
    )
    return f(kern, words)


def _lse_body(x_ref, m_ref, s_ref):
    i = pl.program_id(0)
    x = x_ref[...]
    col = i * LSE_BLK + lax.broadcasted_iota(jnp.int32, x.shape, 1)
    valid = col < VOCAB
    xm = jnp.where(valid, x, -jnp.inf)
    bm = jnp.max(xm, axis=1, keepdims=True)

    @pl.when(i == 0)
    def _():
        m_ref[...] = jnp.full_like(m_ref, -jnp.inf)
        s_ref[...] = jnp.zeros_like(s_ref)

    m_old = m_ref[...]
    m_new = jnp.maximum(m_old, bm)
    bs = jnp.sum(jnp.where(valid, jnp.exp(x - m_new), 0.0), axis=1,
                 keepdims=True)
    s_ref[...] = s_ref[...] * jnp.exp(m_old - m_new) + bs
    m_ref[...] = m_new


def _lse(x):
    return pl.pallas_call(
        _lse_body,
        grid=(LSE_STEPS,),
        in_specs=[pl.BlockSpec((QH, LSE_BLK), lambda i: (0, i))],
        out_specs=[
            pl.BlockSpec((QH, 1), lambda i: (0, 0)),
            pl.BlockSpec((QH, 1), lambda i: (0, 0)),
        ],
        out_shape=[
            jax.ShapeDtypeStruct((QH, 1), jnp.float32),
            jax.ShapeDtypeStruct((QH, 1), jnp.float32),
        ],
    )(x)


def _sub_transpose_a_body(x_ref, m_ref, s_ref, o_ref):
    x = x_ref[...]
    off = m_ref[...] + jnp.log(s_ref[...])
    o_ref[...] = (x - off).T


def _sub_transpose_a(x, m, s):
    vec = pl.BlockSpec((QH, 1), lambda i: (0, 0))
    half = pl.BlockSpec((QH, TR_BLK), lambda i: (0, i))
    return pl.pallas_call(
        _sub_transpose_a_body,
        grid=(TR_STEPS,),
        in_specs=[half, vec, vec],
        out_specs=pl.BlockSpec((TR_BLK, QH), lambda i: (i, 0)),
        out_shape=jax.ShapeDtypeStruct((VOCAB, Q), jnp.float32),
    )(x, m, s)


def _sub_transpose_b_body(prev_ref, x_ref, m_ref, s_ref, o_ref):
    del prev_ref
    x = x_ref[...]
    off = m_ref[...] + jnp.log(s_ref[...])
    o_ref[...] = (x - off).T


def _sub_transpose_b(prev, x, m, s):
    vec = pl.BlockSpec((QH, 1), lambda i: (0, 0))
    half = pl.BlockSpec((QH, TR_BLK), lambda i: (0, i))
    return pl.pallas_call(
        _sub_transpose_b_body,
        grid=(TR_STEPS,),
        in_specs=[
            pl.BlockSpec(memory_space=pl.ANY),
            half, vec, vec,
        ],
        out_specs=pl.BlockSpec((TR_BLK, QH), lambda i: (i, 1)),
        out_shape=jax.ShapeDtypeStruct((VOCAB, Q), jnp.float32),
        input_output_aliases={0: 0},
    )(prev, x, m, s)


def kernel(h_t, cache_hiddens, cache_words):
    kern_a = _kern_matrix(h_t[:QH], cache_hiddens)
    kern_b = _kern_matrix(h_t[QH:], cache_hiddens)
    cp_a = _sc_scatter(kern_a, cache_words)
    cp_b = _sc_scatter(kern_b, cache_words)
    ma, sa = _lse(cp_a)
    out_t = _sub_transpose_a(cp_a, ma, sa)
    cp_b, out_t = lax.optimization_barrier((cp_b, out_t))
    mb, sb = _lse(cp_b)
    out_t = _sub_transpose_b(out_t, cp_b, mb, sb)
    return out_t.T
